# baseline (speedup 1.0000x reference)
import functools

import jax
import jax.numpy as jnp
from jax import lax
from jax.experimental import pallas as pl
from jax.experimental.pallas import tpu as pltpu
from jax.experimental.pallas import tpu_sc as plsc

NC = 2
NS = 16
NW = NC * NS
L = 16


def _make_sc_kernel(B, D, n_ch, ch):
    b_per_w = B // NW
    mesh = plsc.VectorSubcoreMesh(core_axis_name="c", subcore_axis_name="s")

    @functools.partial(
        pl.kernel,
        out_type=jax.ShapeDtypeStruct((NW, b_per_w), jnp.float32),
        mesh=mesh,
        compiler_params=pltpu.CompilerParams(
            needs_layout_passes=False, use_tc_tiling_on_sc=False),
        scratch_types=[
            pltpu.VMEM((n_ch, ch), jnp.int32),
            pltpu.VMEM((n_ch, ch), jnp.int32),
            pltpu.VMEM((b_per_w, D), jnp.float32),
            pltpu.VMEM((b_per_w, D), jnp.float32),
            pltpu.VMEM((b_per_w,), jnp.float32),
            pltpu.SemaphoreType.DMA,
            pltpu.SemaphoreType.DMA,
        ],
    )
    def run(user_hbm, article_hbm, ctab_hbm, atab_hbm, out_hbm,
            idx_c, idx_a, crows, arows, out_v, sem_c, sem_a):
        wid = lax.axis_index("s") * NC + lax.axis_index("c")
        pltpu.sync_copy(user_hbm.at[wid], idx_c)
        pltpu.sync_copy(article_hbm.at[wid], idx_a)
        handles = []
        for j in range(n_ch):
            handles.append(pltpu.async_copy(
                ctab_hbm.at[idx_c.at[j]], crows.at[pl.ds(j * ch, ch)], sem_c))
            handles.append(pltpu.async_copy(
                atab_hbm.at[idx_a.at[j]], arows.at[pl.ds(j * ch, ch)], sem_a))
        for h in handles:
            h.wait()

        def group(g, carry):
            rows = g * L + lax.iota(jnp.int32, L)
            acc = jnp.zeros((L,), jnp.float32)
            for d in range(D):
                col = jnp.full((L,), d, jnp.int32)
                cv = plsc.load_gather(crows, [rows, col])
                av = plsc.load_gather(arows, [rows, col])
                acc = acc + cv * av
            out_v[pl.ds(g * L, L)] = acc
            return carry

        lax.fori_loop(0, b_per_w // L, group, 0)
        pltpu.sync_copy(out_v, out_hbm.at[wid])

    return run


@jax.jit
def kernel(user, article, customer_table, article_table):
    B = user.shape[0]
    D = customer_table.shape[1]
    b_per_w = B // NW
    ch = 128
    n_ch = b_per_w // ch
    user_r = user.reshape(NW, n_ch, ch)
    article_r = article.reshape(NW, n_ch, ch)
    run = _make_sc_kernel(B, D, n_ch, ch)
    out = run(user_r, article_r, customer_table, article_table)
    return out.reshape(B, 1)
